# baseline (speedup 1.0000x reference)
import functools

import jax
import jax.numpy as jnp
from jax import lax
from jax.experimental import pallas as pl
from jax.experimental.pallas import tpu as pltpu
from jax.experimental.pallas import tpu_sc as plsc

N = 10000
E = 160000
D = 256
DHALF = 128
EPS = 1e-5

NUM_CORES = 2
NUM_SUBCORES = 16
EPB = 125
NB = E // NUM_SUBCORES // EPB
CHUNK = 40
ROWS_PER_TILE = N // NUM_SUBCORES
INIT_CHUNK = 5

BR = 2000
GR = N // BR

_DN = (((1,), (1,)), ((), ()))


def _dot_t(a, b):
    return jax.lax.dot_general(a, b, _DN, preferred_element_type=jnp.float32)


@functools.cache
def _make_sc_seg_sum():
    mesh = plsc.VectorSubcoreMesh(core_axis_name="c", subcore_axis_name="s",
                                  num_cores=NUM_CORES)

    @functools.partial(
        pl.kernel,
        out_type=jax.ShapeDtypeStruct((2 * N, DHALF), jnp.float32),
        mesh=mesh,
        scratch_types=[
            pltpu.VMEM((CHUNK, EPB), jnp.int32),
            pltpu.VMEM((CHUNK, EPB), jnp.int32),
            pltpu.VMEM((EPB, DHALF), jnp.float32),
            pltpu.VMEM((EPB, DHALF), jnp.float32),
            pltpu.VMEM_SHARED((N, DHALF), jnp.float32),
            pltpu.SemaphoreType.DMA,
            pltpu.SemaphoreType.DMA,
        ],
        compiler_params=pltpu.CompilerParams(use_tc_tiling_on_sc=False),
    )
    def seg_sum(h_hbm, src_hbm, dst_hbm, out_hbm,
                src_v, dst_v, rows_a, rows_b, acc_sh, sem_a, sem_b):
        c = lax.axis_index("c")
        s = lax.axis_index("s")

        row0 = s * ROWS_PER_TILE

        pltpu.sync_copy(h_hbm.at[pl.ds(c * N + row0, ROWS_PER_TILE)],
                        acc_sh.at[pl.ds(row0, ROWS_PER_TILE)])

        plsc.subcore_barrier()

        b0 = s * NB

        @pl.loop(0, NB, step=CHUNK)
        def _(c0):
            pltpu.sync_copy(
                src_hbm.at[pl.ds(c * (NUM_SUBCORES * NB) + b0 + c0, CHUNK)],
                src_v)
            pltpu.sync_copy(dst_hbm.at[pl.ds(b0 + c0, CHUNK)], dst_v)
            pltpu.async_copy(h_hbm.at[src_v.at[0]], rows_a, sem_a)

            @pl.loop(0, CHUNK, step=2)
            def _(j):
                pltpu.async_copy(h_hbm.at[src_v.at[j + 1]], rows_b, sem_b)
                pltpu.make_async_copy(h_hbm.at[src_v.at[j]], rows_a,
                                      sem_a).wait()
                pltpu.sync_copy(rows_a, acc_sh.at[dst_v.at[j]], add=True)

                @pl.when(j + 2 < CHUNK)
                def _():
                    pltpu.async_copy(h_hbm.at[src_v.at[j + 2]], rows_a, sem_a)

                pltpu.make_async_copy(h_hbm.at[src_v.at[j + 1]], rows_b,
                                      sem_b).wait()
                pltpu.sync_copy(rows_b, acc_sh.at[dst_v.at[j + 1]], add=True)

        plsc.subcore_barrier()

        pltpu.sync_copy(acc_sh.at[pl.ds(row0, ROWS_PER_TILE)],
                        out_hbm.at[pl.ds(c * N + row0, ROWS_PER_TILE)])

    return seg_sum


def _sc_seg_sum(hflat, src2, dst):
    return _make_sc_seg_sum()(hflat, src2, dst)


def _dbn_coeffs(st, g2, be2, go, beo):
    m = st[0:1, :] * (1.0 / N)
    v = st[1:2, :] * (1.0 / N) - m * m
    a1 = g2 / jnp.sqrt(v + EPS)
    c1 = be2 - m * a1
    a2 = go / jnp.sqrt(a1 * a1 * v + EPS)
    c2 = beo - be2 * a2
    return a1 * a2, c1 * a2 + c2


def _bn1_coeffs(st, g, be):
    m = st[0:1, :] * (1.0 / N)
    v = st[1:2, :] * (1.0 / N) - m * m
    a = g / jnp.sqrt(v + EPS)
    return a, be - m * a


def _stats_update(st_ref, y, first):
    st = jax.lax.dot_general(
        jnp.ones((1, BR), jnp.float32), jnp.concatenate([y, y * y], 1),
        (((1,), (0,)), ((), ())), preferred_element_type=jnp.float32,
        precision=jax.lax.Precision.HIGHEST,
    ).reshape(2, D)

    @pl.when(first)
    def _():
        st_ref[...] = st

    @pl.when(jnp.logical_not(first))
    def _():
        st_ref[...] += st


def _layer_body(u_lo, u_hi, w1, b1, g1, be1, w2, b2, g2, be2, go, beo,
                wfc, bfc, out_ref, y1_scr, y2_scr, st1_scr, st2_scr, *, last):
    p = pl.program_id(0)
    g = pl.program_id(1)
    rows = pl.ds(g * BR, BR)

    @pl.when(p == 0)
    def _():
        y = (_dot_t(u_lo[...], w1[:, :DHALF])
             + _dot_t(u_hi[...], w1[:, DHALF:]) + b1[...])
        y1_scr[rows, :] = y
        _stats_update(st1_scr, y, g == 0)

    @pl.when(p == 1)
    def _():
        a, cshift = _bn1_coeffs(st1_scr[...], g1[...], be1[...])
        z = jnp.maximum(y1_scr[rows, :] * a + cshift, 0.0)
        y = _dot_t(z, w2[...]) + b2[...]
        y2_scr[rows, :] = y
        _stats_update(st2_scr, y, g == 0)

    if last:
        @pl.when(p == 2)
        def _():
            a, cshift = _dbn_coeffs(st2_scr[...], g2[...], be2[...],
                                    go[...], beo[...])
            h = jnp.maximum(y2_scr[rows, :] * a + cshift, 0.0)
            out_ref[...] = _dot_t(h, wfc[...]) + bfc[...]
    else:
        @pl.when(p >= 2)
        def _():
            a, cshift = _dbn_coeffs(st2_scr[...], g2[...], be2[...],
                                    go[...], beo[...])
            off = jnp.where(p == 2, 0, DHALF)
            is_lo = p == 2
            ah = jnp.where(is_lo, a[:, :DHALF], a[:, DHALF:])
            ch = jnp.where(is_lo, cshift[:, :DHALF], cshift[:, DHALF:])
            h = jnp.maximum(y2_scr[rows, pl.ds(off, DHALF)] * ah + ch, 0.0)
            out_ref[...] = h


def _tc_layer(u, w1, b1, g1, be1, w2, b2, g2, be2, go, beo, wfc, bfc, last):
    nphase = 3 if last else 4
    const = pl.BlockSpec((1, D), lambda p, g: (0, 0))
    sq = pl.BlockSpec((D, D), lambda p, g: (0, 0))
    if last:
        out_spec = pl.BlockSpec((BR, D),
                                lambda p, g: (jnp.where(p == 2, g, 0), 0))
        out_shape = jax.ShapeDtypeStruct((N, D), jnp.float32)
    else:
        out_spec = pl.BlockSpec(
            (BR, DHALF),
            lambda p, g: (jnp.where(p < 2, 0, (p - 2) * GR + g), 0))
        out_shape = jax.ShapeDtypeStruct((2 * N, DHALF), jnp.float32)
    return pl.pallas_call(
        functools.partial(_layer_body, last=last),
        grid=(nphase, GR),
        in_specs=[
            pl.BlockSpec((BR, DHALF),
                         lambda p, g: (jnp.where(p == 0, g, GR - 1), 0)),
            pl.BlockSpec((BR, DHALF),
                         lambda p, g: (jnp.where(p == 0, g, GR - 1) + GR, 0)),
            sq, const, const, const, sq, const, const, const, const, const,
            sq, const,
        ],
        out_specs=out_spec,
        out_shape=out_shape,
        scratch_shapes=[
            pltpu.VMEM((N, D), jnp.float32),
            pltpu.VMEM((N, D), jnp.float32),
            pltpu.VMEM((2, D), jnp.float32),
            pltpu.VMEM((2, D), jnp.float32),
        ],
    )(u, u, w1, b1, g1, be1, w2, b2, g2, be2, go, beo, wfc, bfc)


def kernel(x, edge_index, params):
    hflat = jnp.concatenate([x[:, :DHALF], x[:, DHALF:]], axis=0)

    src = edge_index[0].reshape(NUM_SUBCORES * NB, EPB)
    dst = edge_index[1].reshape(NUM_SUBCORES * NB, EPB)
    src2 = jnp.concatenate([src, src + N], axis=0)

    row = lambda p: p.reshape(1, D)

    for i in range(3):
        u = _sc_seg_sum(hflat, src2, dst)
        last = i == 2
        hflat = _tc_layer(
            u, params[f"W1_{i}"], row(params[f"b1_{i}"]),
            row(params[f"g1_{i}"]), row(params[f"be1_{i}"]),
            params[f"W2_{i}"], row(params[f"b2_{i}"]),
            row(params[f"g2_{i}"]), row(params[f"be2_{i}"]),
            row(params[f"go_{i}"]), row(params[f"beo_{i}"]),
            params["Wfc"], row(params["bfc"]), last)
    return hflat


# speedup vs baseline: 1.1380x; 1.1380x over previous
import functools

import jax
import jax.numpy as jnp
from jax import lax
from jax.experimental import pallas as pl
from jax.experimental.pallas import tpu as pltpu
from jax.experimental.pallas import tpu_sc as plsc

N = 10000
E = 160000
D = 256
DHALF = 128
EPS = 1e-5

NUM_CORES = 2
NUM_SUBCORES = 16
EPB = 125
NB = E // NUM_SUBCORES // EPB
CHUNK = 40
ROWS_PER_TILE = N // NUM_SUBCORES
INIT_CHUNK = 5

BR = 2000
GR = N // BR

_DN = (((1,), (1,)), ((), ()))


def _dot_t(a, b):
    return jax.lax.dot_general(a, b, _DN, preferred_element_type=jnp.float32)


@functools.cache
def _make_sc_seg_sum():
    mesh = plsc.VectorSubcoreMesh(core_axis_name="c", subcore_axis_name="s",
                                  num_cores=NUM_CORES)

    @functools.partial(
        pl.kernel,
        out_type=jax.ShapeDtypeStruct((2 * N, DHALF), jnp.float32),
        mesh=mesh,
        scratch_types=[
            pltpu.VMEM((CHUNK, EPB), jnp.int32),
            pltpu.VMEM((CHUNK, EPB), jnp.int32),
            pltpu.VMEM((EPB, DHALF), jnp.float32),
            pltpu.VMEM((EPB, DHALF), jnp.float32),
            pltpu.VMEM_SHARED((N, DHALF), jnp.float32),
            pltpu.SemaphoreType.DMA,
            pltpu.SemaphoreType.DMA,
            pltpu.SemaphoreType.DMA,
        ],
        compiler_params=pltpu.CompilerParams(use_tc_tiling_on_sc=False),
    )
    def seg_sum(h_hbm, src_hbm, dst_hbm, out_hbm,
                src_v, dst_v, rows_a, rows_b, acc_sh, sem_a, sem_b, sem_i):
        c = lax.axis_index("c")
        s = lax.axis_index("s")

        row0 = s * ROWS_PER_TILE
        b0 = s * NB

        init_cp = pltpu.async_copy(
            h_hbm.at[pl.ds(c * N + row0, ROWS_PER_TILE)],
            acc_sh.at[pl.ds(row0, ROWS_PER_TILE)], sem_i)
        pltpu.sync_copy(src_hbm.at[pl.ds(c * (NUM_SUBCORES * NB) + b0, CHUNK)],
                        src_v)
        pltpu.sync_copy(dst_hbm.at[pl.ds(b0, CHUNK)], dst_v)
        pltpu.async_copy(h_hbm.at[src_v.at[0]], rows_a, sem_a)
        pltpu.async_copy(h_hbm.at[src_v.at[1]], rows_b, sem_b)
        init_cp.wait()
        plsc.subcore_barrier()

        @pl.loop(0, NB, step=CHUNK)
        def _(c0):
            @pl.when(c0 > 0)
            def _():
                pltpu.sync_copy(
                    src_hbm.at[pl.ds(c * (NUM_SUBCORES * NB) + b0 + c0,
                                     CHUNK)], src_v)
                pltpu.sync_copy(dst_hbm.at[pl.ds(b0 + c0, CHUNK)], dst_v)
                pltpu.async_copy(h_hbm.at[src_v.at[0]], rows_a, sem_a)
                pltpu.async_copy(h_hbm.at[src_v.at[1]], rows_b, sem_b)

            @pl.loop(0, CHUNK, step=2)
            def _(j):
                pltpu.make_async_copy(h_hbm.at[src_v.at[j]], rows_a,
                                      sem_a).wait()
                pltpu.sync_copy(rows_a, acc_sh.at[dst_v.at[j]], add=True)

                @pl.when(j + 2 < CHUNK)
                def _():
                    pltpu.async_copy(h_hbm.at[src_v.at[j + 2]], rows_a, sem_a)

                pltpu.make_async_copy(h_hbm.at[src_v.at[j + 1]], rows_b,
                                      sem_b).wait()
                pltpu.sync_copy(rows_b, acc_sh.at[dst_v.at[j + 1]], add=True)

                @pl.when(j + 3 < CHUNK)
                def _():
                    pltpu.async_copy(h_hbm.at[src_v.at[j + 3]], rows_b, sem_b)

        plsc.subcore_barrier()

        pltpu.sync_copy(acc_sh.at[pl.ds(row0, ROWS_PER_TILE)],
                        out_hbm.at[pl.ds(c * N + row0, ROWS_PER_TILE)])

    return seg_sum


def _sc_seg_sum(hflat, src2, dst):
    return _make_sc_seg_sum()(hflat, src2, dst)


def _dbn_coeffs(st, g2, be2, go, beo):
    m = st[0:1, :] * (1.0 / N)
    v = st[1:2, :] * (1.0 / N) - m * m
    a1 = g2 / jnp.sqrt(v + EPS)
    c1 = be2 - m * a1
    a2 = go / jnp.sqrt(a1 * a1 * v + EPS)
    c2 = beo - be2 * a2
    return a1 * a2, c1 * a2 + c2


def _bn1_coeffs(st, g, be):
    m = st[0:1, :] * (1.0 / N)
    v = st[1:2, :] * (1.0 / N) - m * m
    a = g / jnp.sqrt(v + EPS)
    return a, be - m * a


def _stats_update(st_ref, y, first):
    st = jnp.concatenate([jnp.sum(y, 0, keepdims=True),
                          jnp.sum(y * y, 0, keepdims=True)], 0)

    @pl.when(first)
    def _():
        st_ref[...] = st

    @pl.when(jnp.logical_not(first))
    def _():
        st_ref[...] += st


def _layer_body(u_lo, u_hi, w1, b1, g1, be1, w2, b2, g2, be2, go, beo,
                wfc, bfc, out_ref, y1_scr, y2_scr, st1_scr, st2_scr, *, last):
    p = pl.program_id(0)
    g = pl.program_id(1)
    rows = pl.ds(g * BR, BR)

    @pl.when(p == 0)
    def _():
        y = (_dot_t(u_lo[...], w1[:, :DHALF])
             + _dot_t(u_hi[...], w1[:, DHALF:]) + b1[...])
        y1_scr[rows, :] = y
        _stats_update(st1_scr, y, g == 0)

    @pl.when(p == 1)
    def _():
        a, cshift = _bn1_coeffs(st1_scr[...], g1[...], be1[...])
        z = jnp.maximum(y1_scr[rows, :] * a + cshift, 0.0)
        y = _dot_t(z, w2[...]) + b2[...]
        y2_scr[rows, :] = y
        _stats_update(st2_scr, y, g == 0)

    if last:
        @pl.when(p == 2)
        def _():
            a, cshift = _dbn_coeffs(st2_scr[...], g2[...], be2[...],
                                    go[...], beo[...])
            h = jnp.maximum(y2_scr[rows, :] * a + cshift, 0.0)
            out_ref[...] = _dot_t(h, wfc[...]) + bfc[...]
    else:
        @pl.when(p >= 2)
        def _():
            a, cshift = _dbn_coeffs(st2_scr[...], g2[...], be2[...],
                                    go[...], beo[...])
            off = jnp.where(p == 2, 0, DHALF)
            is_lo = p == 2
            ah = jnp.where(is_lo, a[:, :DHALF], a[:, DHALF:])
            ch = jnp.where(is_lo, cshift[:, :DHALF], cshift[:, DHALF:])
            h = jnp.maximum(y2_scr[rows, pl.ds(off, DHALF)] * ah + ch, 0.0)
            out_ref[...] = h


def _tc_layer(u, w1, b1, g1, be1, w2, b2, g2, be2, go, beo, wfc, bfc, last):
    nphase = 3 if last else 4
    const = pl.BlockSpec((1, D), lambda p, g: (0, 0))
    sq = pl.BlockSpec((D, D), lambda p, g: (0, 0))
    if last:
        out_spec = pl.BlockSpec((BR, D),
                                lambda p, g: (jnp.where(p == 2, g, 0), 0))
        out_shape = jax.ShapeDtypeStruct((N, D), jnp.float32)
    else:
        out_spec = pl.BlockSpec(
            (BR, DHALF),
            lambda p, g: (jnp.where(p < 2, 0, (p - 2) * GR + g), 0))
        out_shape = jax.ShapeDtypeStruct((2 * N, DHALF), jnp.float32)
    return pl.pallas_call(
        functools.partial(_layer_body, last=last),
        grid=(nphase, GR),
        in_specs=[
            pl.BlockSpec((BR, DHALF),
                         lambda p, g: (jnp.where(p == 0, g, GR - 1), 0)),
            pl.BlockSpec((BR, DHALF),
                         lambda p, g: (jnp.where(p == 0, g, GR - 1) + GR, 0)),
            sq, const, const, const, sq, const, const, const, const, const,
            sq, const,
        ],
        out_specs=out_spec,
        out_shape=out_shape,
        scratch_shapes=[
            pltpu.VMEM((N, D), jnp.float32),
            pltpu.VMEM((N, D), jnp.float32),
            pltpu.VMEM((2, D), jnp.float32),
            pltpu.VMEM((2, D), jnp.float32),
        ],
    )(u, u, w1, b1, g1, be1, w2, b2, g2, be2, go, beo, wfc, bfc)


def kernel(x, edge_index, params):
    hflat = jnp.concatenate([x[:, :DHALF], x[:, DHALF:]], axis=0)

    src = edge_index[0].reshape(NUM_SUBCORES * NB, EPB)
    dst = edge_index[1].reshape(NUM_SUBCORES * NB, EPB)
    src2 = jnp.concatenate([src, src + N], axis=0)

    row = lambda p: p.reshape(1, D)

    for i in range(3):
        u = _sc_seg_sum(hflat, src2, dst)
        last = i == 2
        hflat = _tc_layer(
            u, params[f"W1_{i}"], row(params[f"b1_{i}"]),
            row(params[f"g1_{i}"]), row(params[f"be1_{i}"]),
            params[f"W2_{i}"], row(params[f"b2_{i}"]),
            row(params[f"g2_{i}"]), row(params[f"be2_{i}"]),
            row(params[f"go_{i}"]), row(params[f"beo_{i}"]),
            params["Wfc"], row(params["bfc"]), last)
    return hflat


# speedup vs baseline: 1.1410x; 1.0027x over previous
import functools

import jax
import jax.numpy as jnp
from jax import lax
from jax.experimental import pallas as pl
from jax.experimental.pallas import tpu as pltpu
from jax.experimental.pallas import tpu_sc as plsc

N = 10000
E = 160000
D = 256
DHALF = 128
EPS = 1e-5

NUM_CORES = 2
NUM_SUBCORES = 16
EPB = 125
NB = E // NUM_SUBCORES // EPB
CHUNK = 40
ROWS_PER_TILE = N // NUM_SUBCORES

BR = 2000
GR = N // BR

_DN = (((1,), (1,)), ((), ()))


def _dot_t(a, b):
    return jax.lax.dot_general(a, b, _DN, preferred_element_type=jnp.float32)


@functools.cache
def _make_sc_seg_sum():
    mesh = plsc.VectorSubcoreMesh(core_axis_name="c", subcore_axis_name="s",
                                  num_cores=NUM_CORES)

    @functools.partial(
        pl.kernel,
        out_type=jax.ShapeDtypeStruct((2 * N, DHALF), jnp.float32),
        mesh=mesh,
        scratch_types=[
            pltpu.VMEM((CHUNK, EPB), jnp.int32),
            pltpu.VMEM((CHUNK, EPB), jnp.int32),
            pltpu.VMEM((EPB, DHALF), jnp.float32),
            pltpu.VMEM((EPB, DHALF), jnp.float32),
            pltpu.VMEM_SHARED((N, DHALF), jnp.float32),
            pltpu.SemaphoreType.DMA,
            pltpu.SemaphoreType.DMA,
            pltpu.SemaphoreType.DMA,
        ],
        compiler_params=pltpu.CompilerParams(use_tc_tiling_on_sc=False),
    )
    def seg_sum(h_hbm, src_hbm, dst_hbm, out_hbm,
                src_v, dst_v, rows_a, rows_b, acc_sh, sem_a, sem_b, sem_i):
        c = lax.axis_index("c")
        s = lax.axis_index("s")

        row0 = s * ROWS_PER_TILE
        b0 = s * NB

        init_cp = pltpu.async_copy(
            h_hbm.at[pl.ds(c * N + row0, ROWS_PER_TILE)],
            acc_sh.at[pl.ds(row0, ROWS_PER_TILE)], sem_i)
        pltpu.sync_copy(src_hbm.at[pl.ds(c * (NUM_SUBCORES * NB) + b0, CHUNK)],
                        src_v)
        pltpu.sync_copy(dst_hbm.at[pl.ds(b0, CHUNK)], dst_v)
        pltpu.async_copy(h_hbm.at[src_v.at[0]], rows_a, sem_a)
        pltpu.async_copy(h_hbm.at[src_v.at[1]], rows_b, sem_b)
        init_cp.wait()
        plsc.subcore_barrier()

        @pl.loop(0, NB, step=CHUNK)
        def _(c0):
            @pl.when(c0 > 0)
            def _():
                pltpu.sync_copy(
                    src_hbm.at[pl.ds(c * (NUM_SUBCORES * NB) + b0 + c0,
                                     CHUNK)], src_v)
                pltpu.sync_copy(dst_hbm.at[pl.ds(b0 + c0, CHUNK)], dst_v)
                pltpu.async_copy(h_hbm.at[src_v.at[0]], rows_a, sem_a)
                pltpu.async_copy(h_hbm.at[src_v.at[1]], rows_b, sem_b)

            @pl.loop(0, CHUNK, step=2)
            def _(j):
                pltpu.make_async_copy(h_hbm.at[src_v.at[j]], rows_a,
                                      sem_a).wait()
                pltpu.sync_copy(rows_a, acc_sh.at[dst_v.at[j]], add=True)

                @pl.when(j + 2 < CHUNK)
                def _():
                    pltpu.async_copy(h_hbm.at[src_v.at[j + 2]], rows_a, sem_a)

                pltpu.make_async_copy(h_hbm.at[src_v.at[j + 1]], rows_b,
                                      sem_b).wait()
                pltpu.sync_copy(rows_b, acc_sh.at[dst_v.at[j + 1]], add=True)

                @pl.when(j + 3 < CHUNK)
                def _():
                    pltpu.async_copy(h_hbm.at[src_v.at[j + 3]], rows_b, sem_b)

        plsc.subcore_barrier()

        pltpu.sync_copy(acc_sh.at[pl.ds(row0, ROWS_PER_TILE)],
                        out_hbm.at[pl.ds(c * N + row0, ROWS_PER_TILE)])

    return seg_sum


def _sc_seg_sum(hflat, src2, dst):
    return _make_sc_seg_sum()(hflat, src2, dst)


def _dbn_coeffs(st, g2, be2, go, beo):
    m = st[0:1, :] * (1.0 / N)
    v = st[1:2, :] * (1.0 / N) - m * m
    a1 = g2 / jnp.sqrt(v + EPS)
    c1 = be2 - m * a1
    a2 = go / jnp.sqrt(a1 * a1 * v + EPS)
    c2 = beo - be2 * a2
    return a1 * a2, c1 * a2 + c2


def _bn1_coeffs(st, g, be):
    m = st[0:1, :] * (1.0 / N)
    v = st[1:2, :] * (1.0 / N) - m * m
    a = g / jnp.sqrt(v + EPS)
    return a, be - m * a


def _stats_update(st_ref, y, first):
    st = jnp.concatenate([jnp.sum(y, 0, keepdims=True),
                          jnp.sum(y * y, 0, keepdims=True)], 0)

    @pl.when(first)
    def _():
        st_ref[...] = st

    @pl.when(jnp.logical_not(first))
    def _():
        st_ref[...] += st


def _layer_body(u_lo, u_hi, w1, b1, g1, be1, w2, b2, g2, be2, go, beo,
                wfc, bfc, out_ref, y1_scr, y2_scr, st1_scr, st2_scr, *, last):
    p = pl.program_id(0)
    g = pl.program_id(1)
    rows = pl.ds(g * BR, BR)

    @pl.when(p == 0)
    def _():
        y = (_dot_t(u_lo[...], w1[:, :DHALF])
             + _dot_t(u_hi[...], w1[:, DHALF:]) + b1[...])
        y1_scr[rows, :] = y
        _stats_update(st1_scr, y, g == 0)

    @pl.when(p == 1)
    def _():
        a, cshift = _bn1_coeffs(st1_scr[...], g1[...], be1[...])
        z = jnp.maximum(y1_scr[rows, :] * a + cshift, 0.0)
        y = _dot_t(z, w2[...]) + b2[...]
        y2_scr[rows, :] = y
        _stats_update(st2_scr, y, g == 0)

    if last:
        @pl.when(p == 2)
        def _():
            a, cshift = _dbn_coeffs(st2_scr[...], g2[...], be2[...],
                                    go[...], beo[...])
            h = jnp.maximum(y2_scr[rows, :] * a + cshift, 0.0)
            out_ref[...] = _dot_t(h, wfc[...]) + bfc[...]
    else:
        @pl.when(p >= 2)
        def _():
            a, cshift = _dbn_coeffs(st2_scr[...], g2[...], be2[...],
                                    go[...], beo[...])
            off = jnp.where(p == 2, 0, DHALF)
            is_lo = p == 2
            ah = jnp.where(is_lo, a[:, :DHALF], a[:, DHALF:])
            ch = jnp.where(is_lo, cshift[:, :DHALF], cshift[:, DHALF:])
            h = jnp.maximum(y2_scr[rows, pl.ds(off, DHALF)] * ah + ch, 0.0)
            out_ref[...] = h


def _tc_layer(u, w1, b1, g1, be1, w2, b2, g2, be2, go, beo, wfc, bfc, last):
    nphase = 3 if last else 4
    const = pl.BlockSpec((1, D), lambda p, g: (0, 0))
    sq = pl.BlockSpec((D, D), lambda p, g: (0, 0))
    if last:
        out_spec = pl.BlockSpec((BR, D),
                                lambda p, g: (jnp.where(p == 2, g, 0), 0))
        out_shape = jax.ShapeDtypeStruct((N, D), jnp.float32)
    else:
        out_spec = pl.BlockSpec(
            (BR, DHALF),
            lambda p, g: (jnp.where(p < 2, 0, (p - 2) * GR + g), 0))
        out_shape = jax.ShapeDtypeStruct((2 * N, DHALF), jnp.float32)
    return pl.pallas_call(
        functools.partial(_layer_body, last=last),
        grid=(nphase, GR),
        in_specs=[
            pl.BlockSpec((BR, DHALF),
                         lambda p, g: (jnp.where(p == 0, g, GR - 1), 0)),
            pl.BlockSpec((BR, DHALF),
                         lambda p, g: (jnp.where(p == 0, g, GR - 1) + GR, 0)),
            sq, const, const, const, sq, const, const, const, const, const,
            sq, const,
        ],
        out_specs=out_spec,
        out_shape=out_shape,
        scratch_shapes=[
            pltpu.VMEM((N, D), jnp.float32),
            pltpu.VMEM((N, D), jnp.float32),
            pltpu.VMEM((2, D), jnp.float32),
            pltpu.VMEM((2, D), jnp.float32),
        ],
    )(u, u, w1, b1, g1, be1, w2, b2, g2, be2, go, beo, wfc, bfc)


def kernel(x, edge_index, params):
    hflat = jnp.concatenate([x[:, :DHALF], x[:, DHALF:]], axis=0)

    src = edge_index[0].reshape(NUM_SUBCORES * NB, EPB)
    dst = edge_index[1].reshape(NUM_SUBCORES * NB, EPB)
    src2 = jnp.concatenate([src, src + N], axis=0)

    row = lambda p: p.reshape(1, D)

    for i in range(3):
        u = _sc_seg_sum(hflat, src2, dst)
        last = i == 2
        hflat = _tc_layer(
            u, params[f"W1_{i}"], row(params[f"b1_{i}"]),
            row(params[f"g1_{i}"]), row(params[f"be1_{i}"]),
            params[f"W2_{i}"], row(params[f"b2_{i}"]),
            row(params[f"g2_{i}"]), row(params[f"be2_{i}"]),
            row(params[f"go_{i}"]), row(params[f"beo_{i}"]),
            params["Wfc"], row(params["bfc"]), last)
    return hflat
